# Initial kernel scaffold; baseline (speedup 1.0000x reference)
import jax
import jax.numpy as jnp
from jax.experimental import pallas as pl


def kernel(x, W_off, b_off, W_dw, b_dw, W_pw, b_pw):
    raise NotImplementedError("write your pallas kernel here")


# speedup vs baseline: 3.5356x; 3.5356x over previous
import functools

import jax
import jax.numpy as jnp
from jax import lax
from jax.experimental import pallas as pl
from jax.experimental.pallas import tpu as pltpu
from jax.experimental.pallas import tpu_sc as plsc

H = 224
W = 224
C = 96
K = 3
KK = K * K
F = 96
N = H * W * KK

HB1 = 16
HB3 = 8
NW = 32
SPW = N // NW
SCH = 144
CP = 128
NCH = SPW // SCH


def _copy_halo_rows(src_any, halo_ref, sem, r, hb, nrows, row_mask_zero):
    top = jnp.maximum(r * hb - 1, 0)
    bot = jnp.minimum(r * hb + hb, nrows - 1)
    cp0 = pltpu.make_async_copy(src_any.at[pl.ds(top, 1)], halo_ref.at[pl.ds(0, 1)], sem)
    cp0.start()
    cp0.wait()
    cp1 = pltpu.make_async_copy(src_any.at[pl.ds(bot, 1)], halo_ref.at[pl.ds(1, 1)], sem)
    cp1.start()
    cp1.wait()
    nblk = nrows // hb
    tmask = jnp.where(r == 0, 0.0, 1.0)
    bmask = jnp.where(r == nblk - 1, 0.0, 1.0)
    return tmask, bmask


def _shift_w(a, dx):
    hb, w, ch = a.shape
    z = jnp.zeros((hb, 1, ch), a.dtype)
    if dx == -1:
        return jnp.concatenate([z, a[:, : w - 1, :]], axis=1)
    if dx == 1:
        return jnp.concatenate([a[:, 1:, :], z], axis=1)
    return a


def _stage1_body(x_blk, x_any, wofft, boff2, ia, ib, ic, id_, wa, wb, wc, wd,
                 halo, sem):
    r = pl.program_id(0)
    tmask, bmask = _copy_halo_rows(x_any, halo, sem, r, HB1, H, True)
    xb = x_blk[...]
    top = (halo[0] * tmask)[None]
    bot = (halo[1] * bmask)[None]
    rows = jnp.concatenate([top, xb, bot], axis=0)

    off = jnp.zeros((HB1 * W, 2 * KK), jnp.float32)
    for ky in range(3):
        sl = rows[ky:ky + HB1]
        for kx in range(3):
            sh = _shift_w(sl, kx - 1).reshape(HB1 * W, C)
            off = off + jnp.dot(sh, wofft[ky * 3 + kx],
                                preferred_element_type=jnp.float32)
    off = off + boff2[0]
    off3 = off.reshape(HB1, W, 2 * KK)
    ox = off3[:, :, :KK]
    oy = off3[:, :, KK:]

    gx = lax.broadcasted_iota(jnp.int32, (HB1, W, KK), 1).astype(jnp.float32)
    gy = (lax.broadcasted_iota(jnp.int32, (HB1, W, KK), 0)
          + r * HB1).astype(jnp.float32)
    kkf = lax.broadcasted_iota(jnp.int32, (HB1, W, KK), 2).astype(jnp.float32)
    t = jnp.floor(kkf / 3.0)
    dyk = t - 1.0
    dxk = kkf - 3.0 * t - 1.0

    lx = jnp.clip(gx + dxk + ox, 0.0, W - 1.0)
    ly = jnp.clip(gy + dyk + oy, 0.0, H - 1.0)
    x0 = jnp.floor(lx)
    x1 = x0 + 1.0
    y0 = jnp.floor(ly)
    y1 = y0 + 1.0
    x0 = jnp.clip(x0, 0.0, W - 1.0)
    x1 = jnp.clip(x1, 0.0, W - 1.0)
    y0 = jnp.clip(y0, 0.0, H - 1.0)
    y1 = jnp.clip(y1, 0.0, H - 1.0)
    wa[...] = (x1 - lx) * (y1 - ly)
    wb[...] = (x1 - lx) * (ly - y0)
    wc[...] = (lx - x0) * (y1 - ly)
    wd[...] = (lx - x0) * (ly - y0)
    x0i = x0.astype(jnp.int32)
    x1i = x1.astype(jnp.int32)
    y0i = y0.astype(jnp.int32)
    y1i = y1.astype(jnp.int32)
    ia[...] = y0i * W + x0i
    ib[...] = y1i * W + x0i
    ic[...] = y0i * W + x1i
    id_[...] = y1i * W + x1i


def _stage1(x2, wofft, boff2):
    grid = H // HB1
    idx_like = jax.ShapeDtypeStruct((H, W, KK), jnp.int32)
    w_like = jax.ShapeDtypeStruct((H, W, KK), jnp.float32)
    out_spec = pl.BlockSpec((HB1, W, KK), lambda r: (r, 0, 0))
    return pl.pallas_call(
        _stage1_body,
        grid=(grid,),
        in_specs=[
            pl.BlockSpec((HB1, W, C), lambda r: (r, 0, 0)),
            pl.BlockSpec(memory_space=pl.ANY),
            pl.BlockSpec((KK, C, 2 * KK), lambda r: (0, 0, 0)),
            pl.BlockSpec((1, 2 * KK), lambda r: (0, 0)),
        ],
        out_specs=[out_spec] * 4 + [out_spec] * 4,
        out_shape=[idx_like] * 4 + [w_like] * 4,
        scratch_shapes=[
            pltpu.VMEM((2, W, C), jnp.float32),
            pltpu.SemaphoreType.DMA,
        ],
    )(x2, x2, wofft, boff2)


def _sc_body(xflat, ia, ib, ic, id_, wa, wb, wc, wd, out,
             iav, ibv, icv, idv, wav, wbv, wcv, wdv,
             abuf, bbuf, cbuf, dbuf, obuf, sem):
    cid = lax.axis_index("c")
    sid = lax.axis_index("s")
    wid = sid * 2 + cid
    base = wid * SPW

    def chunk(j, carry):
        s0 = base + j * SCH
        pltpu.sync_copy(ia.at[pl.ds(s0, SCH)], iav)
        pltpu.sync_copy(ib.at[pl.ds(s0, SCH)], ibv)
        pltpu.sync_copy(ic.at[pl.ds(s0, SCH)], icv)
        pltpu.sync_copy(id_.at[pl.ds(s0, SCH)], idv)
        pltpu.sync_copy(wa.at[pl.ds(s0, SCH)], wav)
        pltpu.sync_copy(wb.at[pl.ds(s0, SCH)], wbv)
        pltpu.sync_copy(wc.at[pl.ds(s0, SCH)], wcv)
        pltpu.sync_copy(wd.at[pl.ds(s0, SCH)], wdv)
        cpa = pltpu.make_async_copy(xflat.at[iav], abuf, sem)
        cpb = pltpu.make_async_copy(xflat.at[ibv], bbuf, sem)
        cpc = pltpu.make_async_copy(xflat.at[icv], cbuf, sem)
        cpd = pltpu.make_async_copy(xflat.at[idv], dbuf, sem)
        cpa.start()
        cpb.start()
        cpc.start()
        cpd.start()
        cpa.wait()
        cpb.wait()
        cpc.wait()
        cpd.wait()

        def samp16(b, carry2):
            b16 = b * 16
            wavec = wav[pl.ds(b16, 16)]
            wbvec = wbv[pl.ds(b16, 16)]
            wcvec = wcv[pl.ds(b16, 16)]
            wdvec = wdv[pl.ds(b16, 16)]
            for i in range(16):
                s = b16 + i
                va = jnp.full((16,), wavec[i], jnp.float32)
                vb = jnp.full((16,), wbvec[i], jnp.float32)
                vc = jnp.full((16,), wcvec[i], jnp.float32)
                vd = jnp.full((16,), wdvec[i], jnp.float32)
                for cb in range(C // 16):
                    sl = pl.ds(cb * 16, 16)
                    obuf[s, sl] = (va * abuf[s, sl] + vb * bbuf[s, sl]
                                   + vc * cbuf[s, sl] + vd * dbuf[s, sl])
            return carry2

        lax.fori_loop(0, SCH // 16, samp16, 0)
        pltpu.sync_copy(obuf, out.at[pl.ds(s0, SCH)])
        return carry

    lax.fori_loop(0, NCH, chunk, 0)


def _stage2(xflat, ia, ib, ic, id_, wa, wb, wc, wd):
    mesh = plsc.VectorSubcoreMesh(core_axis_name="c", subcore_axis_name="s")
    f = functools.partial(
        pl.kernel,
        out_type=jax.ShapeDtypeStruct((N, C), jnp.float32),
        mesh=mesh,
        scratch_types=[
            pltpu.VMEM((SCH,), jnp.int32),
            pltpu.VMEM((SCH,), jnp.int32),
            pltpu.VMEM((SCH,), jnp.int32),
            pltpu.VMEM((SCH,), jnp.int32),
            pltpu.VMEM((SCH,), jnp.float32),
            pltpu.VMEM((SCH,), jnp.float32),
            pltpu.VMEM((SCH,), jnp.float32),
            pltpu.VMEM((SCH,), jnp.float32),
            pltpu.VMEM((SCH, CP), jnp.float32),
            pltpu.VMEM((SCH, CP), jnp.float32),
            pltpu.VMEM((SCH, CP), jnp.float32),
            pltpu.VMEM((SCH, CP), jnp.float32),
            pltpu.VMEM((SCH, C), jnp.float32),
            pltpu.SemaphoreType.DMA,
        ],
    )(_sc_body)
    return f(xflat, ia, ib, ic, id_, wa, wb, wc, wd)


def _stage3_body(s_blk, s_any, wdw, bdw, wpw, bpw, out, halo, sem):
    r = pl.program_id(0)
    tmask, bmask = _copy_halo_rows(s_any, halo, sem, r, HB3, H, True)
    sb = s_blk[...]
    top = (halo[0] * tmask)[None]
    bot = (halo[1] * bmask)[None]
    rows = jnp.concatenate([top, sb, bot], axis=0)

    acc = jnp.broadcast_to(bdw[0], (HB3, W, KK * C))
    for ky in range(3):
        sl = rows[ky:ky + HB3]
        for kx in range(3):
            acc = acc + _shift_w(sl, kx - 1) * wdw[ky * 3 + kx]
    y = jnp.dot(acc.reshape(HB3 * W, KK * C), wpw[...],
                preferred_element_type=jnp.float32) + bpw[0]
    out[...] = y


def _stage3(samp3, wdw, bdw, wpw, bpw):
    grid = H // HB3
    return pl.pallas_call(
        _stage3_body,
        grid=(grid,),
        in_specs=[
            pl.BlockSpec((HB3, W, KK * C), lambda r: (r, 0, 0)),
            pl.BlockSpec(memory_space=pl.ANY),
            pl.BlockSpec((KK, KK * C), lambda r: (0, 0)),
            pl.BlockSpec((1, KK * C), lambda r: (0, 0)),
            pl.BlockSpec((KK * C, F), lambda r: (0, 0)),
            pl.BlockSpec((1, F), lambda r: (0, 0)),
        ],
        out_specs=pl.BlockSpec((HB3 * W, F), lambda r: (r, 0)),
        out_shape=jax.ShapeDtypeStruct((H * W, F), jnp.float32),
        scratch_shapes=[
            pltpu.VMEM((2, W, KK * C), jnp.float32),
            pltpu.SemaphoreType.DMA,
        ],
    )(samp3, samp3, wdw, bdw, wpw, bpw)


def kernel(x, W_off, b_off, W_dw, b_dw, W_pw, b_pw):
    x2 = x.reshape(H, W, C)
    wf = W_off.reshape(KK, C, 2 * KK)
    wofft = jnp.concatenate([wf[:, :, 0::2], wf[:, :, 1::2]], axis=-1)
    boff2 = jnp.concatenate([b_off[0::2], b_off[1::2]]).reshape(1, 2 * KK)

    ia, ib, ic, id_, wa, wb, wc, wd = _stage1(x2, wofft, boff2)

    xpad = jnp.pad(x.reshape(H * W, C), ((0, 0), (0, CP - C)))
    flat = lambda a: a.reshape(N)
    samp = _stage2(xpad, flat(ia), flat(ib), flat(ic), flat(id_),
                   flat(wa), flat(wb), flat(wc), flat(wd))

    samp3 = samp.reshape(H, W, KK * C)
    wdw = W_dw.reshape(KK, KK * C)
    bdw = b_dw.reshape(1, KK * C)
    wpw = W_pw.reshape(KK * C, F)
    bpw = b_pw.reshape(1, F)
    y = _stage3(samp3, wdw, bdw, wpw, bpw)
    return y.reshape(1, H, W, F)


# speedup vs baseline: 4.6771x; 1.3229x over previous
import functools

import jax
import jax.numpy as jnp
from jax import lax
from jax.experimental import pallas as pl
from jax.experimental.pallas import tpu as pltpu
from jax.experimental.pallas import tpu_sc as plsc

H = 224
W = 224
C = 96
K = 3
KK = K * K
F = 96
N = H * W * KK

HB1 = 16
HB3 = 8
NW = 32
CP = 128
NR = N // 128
SB = HB1 * W * KK
SR = SB // 128
US = 64


def _copy_halo_rows(src_any, halo_ref, sem, r, hb, nrows):
    top = jnp.maximum(r * hb - 1, 0)
    bot = jnp.minimum(r * hb + hb, nrows - 1)
    cp0 = pltpu.make_async_copy(src_any.at[pl.ds(top, 1)], halo_ref.at[pl.ds(0, 1)], sem)
    cp0.start()
    cp0.wait()
    cp1 = pltpu.make_async_copy(src_any.at[pl.ds(bot, 1)], halo_ref.at[pl.ds(1, 1)], sem)
    cp1.start()
    cp1.wait()
    nblk = nrows // hb
    tmask = jnp.where(r == 0, 0.0, 1.0)
    bmask = jnp.where(r == nblk - 1, 0.0, 1.0)
    return tmask, bmask


def _shift_w(a, dx):
    hb, w, ch = a.shape
    z = jnp.zeros((hb, 1, ch), a.dtype)
    if dx == -1:
        return jnp.concatenate([z, a[:, : w - 1, :]], axis=1)
    if dx == 1:
        return jnp.concatenate([a[:, 1:, :], z], axis=1)
    return a


def _stage1_body(x_blk, x_any, wofft, boff2, ia, ib, ic, id_, wa, wb, wc, wd,
                 xpad, halo, sem):
    r = pl.program_id(0)
    tmask, bmask = _copy_halo_rows(x_any, halo, sem, r, HB1, H)
    xb = x_blk[...]
    top = (halo[0] * tmask)[None]
    bot = (halo[1] * bmask)[None]
    rows = jnp.concatenate([top, xb, bot], axis=0)

    off = jnp.zeros((HB1 * W, 2 * KK), jnp.float32)
    for ky in range(3):
        sl = rows[ky:ky + HB1]
        for kx in range(3):
            sh = _shift_w(sl, kx - 1).reshape(HB1 * W, C)
            off = off + jnp.dot(sh, wofft[ky * 3 + kx],
                                preferred_element_type=jnp.float32)
    off = off + boff2[0]
    off3 = off.reshape(HB1, W, 2 * KK)
    ox = off3[:, :, :KK]
    oy = off3[:, :, KK:]

    gx = lax.broadcasted_iota(jnp.int32, (HB1, W, KK), 1).astype(jnp.float32)
    gy = (lax.broadcasted_iota(jnp.int32, (HB1, W, KK), 0)
          + r * HB1).astype(jnp.float32)
    kkf = lax.broadcasted_iota(jnp.int32, (HB1, W, KK), 2).astype(jnp.float32)
    t = jnp.floor(kkf / 3.0)
    dyk = t - 1.0
    dxk = kkf - 3.0 * t - 1.0

    lx = jnp.clip(gx + dxk + ox, 0.0, W - 1.0)
    ly = jnp.clip(gy + dyk + oy, 0.0, H - 1.0)
    x0 = jnp.floor(lx)
    x1 = x0 + 1.0
    y0 = jnp.floor(ly)
    y1 = y0 + 1.0
    x0 = jnp.clip(x0, 0.0, W - 1.0)
    x1 = jnp.clip(x1, 0.0, W - 1.0)
    y0 = jnp.clip(y0, 0.0, H - 1.0)
    y1 = jnp.clip(y1, 0.0, H - 1.0)
    wa[...] = (x1 - lx) * (y1 - ly)
    wb[...] = (x1 - lx) * (ly - y0)
    wc[...] = (lx - x0) * (y1 - ly)
    wd[...] = (lx - x0) * (ly - y0)
    x0i = x0.astype(jnp.int32)
    x1i = x1.astype(jnp.int32)
    y0i = y0.astype(jnp.int32)
    y1i = y1.astype(jnp.int32)
    ia[...] = y0i * W + x0i
    ib[...] = y1i * W + x0i
    ic[...] = y0i * W + x1i
    id_[...] = y1i * W + x1i
    xpad[...] = jnp.concatenate(
        [xb.reshape(HB1 * W, C), jnp.zeros((HB1 * W, CP - C), jnp.float32)],
        axis=1)


def _stage1(x2, wofft, boff2):
    grid = H // HB1
    idx_like = jax.ShapeDtypeStruct((H, W, KK), jnp.int32)
    w_like = jax.ShapeDtypeStruct((H, W, KK), jnp.float32)
    out_spec = pl.BlockSpec((HB1, W, KK), lambda r: (r, 0, 0))
    return pl.pallas_call(
        _stage1_body,
        grid=(grid,),
        in_specs=[
            pl.BlockSpec((HB1, W, C), lambda r: (r, 0, 0)),
            pl.BlockSpec(memory_space=pl.ANY),
            pl.BlockSpec((KK, C, 2 * KK), lambda r: (0, 0, 0)),
            pl.BlockSpec((1, 2 * KK), lambda r: (0, 0)),
        ],
        out_specs=[out_spec] * 8 + [pl.BlockSpec((HB1 * W, CP), lambda r: (r, 0))],
        out_shape=[idx_like] * 4 + [w_like] * 4
        + [jax.ShapeDtypeStruct((H * W, CP), jnp.float32)],
        scratch_shapes=[
            pltpu.VMEM((2, W, C), jnp.float32),
            pltpu.SemaphoreType.DMA,
        ],
    )(x2, x2, wofft, boff2)


def _fire_gathers(xflat, pki, half, bufs, sem):
    for q, buf in enumerate(bufs):
        pltpu.make_async_copy(
            xflat.at[pki.at[q, pl.ds(half * US, US)]], buf, sem).start()


def _wait_gathers(xflat, pki, half, bufs, sem):
    for q, buf in enumerate(bufs):
        pltpu.make_async_copy(
            xflat.at[pki.at[q, pl.ds(half * US, US)]], buf, sem).wait()


def _row_copies(idx8, row, pki, pkw, sem):
    cps = []
    for q in range(4):
        cps.append(pltpu.make_async_copy(
            idx8[q].at[pl.ds(row * 128, 128)], pki.at[q], sem))
    for q in range(4):
        cps.append(pltpu.make_async_copy(
            idx8[4 + q].at[pl.ds(row * 128, 128)], pkw.at[q], sem))
    return cps


def _fetch_row(idx8, row, pki, pkw, sem):
    for cp in _row_copies(idx8, row, pki, pkw, sem):
        cp.start()


def _wait_row(idx8, row, pki, pkw, sem):
    for cp in _row_copies(idx8, row, pki, pkw, sem):
        cp.wait()


def _compute_unit(pkw, half, bufs, ob):
    a4, b4, c4, d4 = bufs

    def blk(tb, carry):
        b16 = tb * 16
        lane = pl.ds(half * US + b16, 16)
        wav = pkw[0, lane]
        wbv = pkw[1, lane]
        wcv = pkw[2, lane]
        wdv = pkw[3, lane]
        for i in range(16):
            srow = b16 + i
            va = jnp.full((16,), wav[i], jnp.float32)
            vb = jnp.full((16,), wbv[i], jnp.float32)
            vc = jnp.full((16,), wcv[i], jnp.float32)
            vd = jnp.full((16,), wdv[i], jnp.float32)
            for cb in range(C // 16):
                slc = pl.ds(cb * 16, 16)
                ob[srow, slc] = (va * a4[srow, slc] + vb * b4[srow, slc]
                                 + vc * c4[srow, slc] + vd * d4[srow, slc])
        return carry

    lax.fori_loop(0, US // 16, blk, 0)


def _sc_body(xflat, ia, ib, ic, id_, wa, wb, wc, wd, out,
             pi0, pi1, pw0, pw1, a0, b0, c0, d0, a1, b1, c1, d1, o0, o1,
             sgat0, sgat1, sout0, sout1, sidx0, sidx1):
    cid = lax.axis_index("c")
    sid = lax.axis_index("s")
    wid = sid * 2 + cid
    r0w = wid * 110 + 2 * jnp.minimum(wid, 4)
    nr = jnp.where(wid < 4, 112, 110)

    idx8 = (ia, ib, ic, id_, wa, wb, wc, wd)
    pki = (pi0, pi1)
    pkw = (pw0, pw1)
    gb = ((a0, b0, c0, d0), (a1, b1, c1, d1))
    ob = (o0, o1)
    sgat = (sgat0, sgat1)
    sout = (sout0, sout1)
    sidx = (sidx0, sidx1)

    _fetch_row(idx8, r0w, pi0, pw0, sidx0)
    _wait_row(idx8, r0w, pi0, pw0, sidx0)
    _fire_gathers(xflat, pi0, 0, gb[0], sgat0)

    @pl.loop(0, nr, step=2)
    def _row_pair(rl):
        for rp in (0, 1):
            ar = r0w + rl + rp
            for b in (0, 1):
                _wait_gathers(xflat, pki[rp], b, gb[b], sgat[b])
                if b == 0:
                    @pl.when(rl + rp + 1 < nr)
                    def _():
                        _fetch_row(idx8, ar + 1, pki[rp ^ 1], pkw[rp ^ 1],
                                   sidx[rp ^ 1])
                    _fire_gathers(xflat, pki[rp], 1, gb[1], sgat[1])
                else:
                    @pl.when(rl + rp + 1 < nr)
                    def _():
                        _wait_row(idx8, ar + 1, pki[rp ^ 1], pkw[rp ^ 1],
                                  sidx[rp ^ 1])
                        _fire_gathers(xflat, pki[rp ^ 1], 0, gb[0], sgat[0])

                @pl.when(rl + rp >= 1)
                def _():
                    pltpu.make_async_copy(
                        ob[b], out.at[pl.ds(0, US)], sout[b]).wait()
                _compute_unit(pkw[rp], b, gb[b], ob[b])
                pltpu.make_async_copy(
                    ob[b], out.at[pl.ds(ar * 128 + b * US, US)], sout[b]).start()

    for b in (0, 1):
        pltpu.make_async_copy(ob[b], out.at[pl.ds(0, US)], sout[b]).wait()


def _stage2(xflat, ia, ib, ic, id_, wa, wb, wc, wd):
    mesh = plsc.VectorSubcoreMesh(core_axis_name="c", subcore_axis_name="s")
    f = functools.partial(
        pl.kernel,
        out_type=jax.ShapeDtypeStruct((N, CP), jnp.float32),
        mesh=mesh,
        scratch_types=[
            pltpu.VMEM((4, 128), jnp.int32),
            pltpu.VMEM((4, 128), jnp.int32),
            pltpu.VMEM((4, 128), jnp.float32),
            pltpu.VMEM((4, 128), jnp.float32),
            pltpu.VMEM((US, CP), jnp.float32),
            pltpu.VMEM((US, CP), jnp.float32),
            pltpu.VMEM((US, CP), jnp.float32),
            pltpu.VMEM((US, CP), jnp.float32),
            pltpu.VMEM((US, CP), jnp.float32),
            pltpu.VMEM((US, CP), jnp.float32),
            pltpu.VMEM((US, CP), jnp.float32),
            pltpu.VMEM((US, CP), jnp.float32),
            pltpu.VMEM((US, CP), jnp.float32),
            pltpu.VMEM((US, CP), jnp.float32),
            pltpu.SemaphoreType.DMA,
            pltpu.SemaphoreType.DMA,
            pltpu.SemaphoreType.DMA,
            pltpu.SemaphoreType.DMA,
            pltpu.SemaphoreType.DMA,
            pltpu.SemaphoreType.DMA,
        ],
    )(_sc_body)
    return f(xflat, ia, ib, ic, id_, wa, wb, wc, wd)


def _stage3_body(s_blk, s_any, wdw, bdw, wpw, bpw, out, halo, sem):
    r = pl.program_id(0)
    tmask, bmask = _copy_halo_rows(s_any, halo, sem, r, HB3, H)
    sb = s_blk[...]
    top = (halo[0] * tmask)[None]
    bot = (halo[1] * bmask)[None]
    rows = jnp.concatenate([top, sb, bot], axis=0)

    acc = jnp.broadcast_to(bdw[0], (HB3, W, KK * C))
    for ky in range(3):
        sl = rows[ky:ky + HB3]
        for kx in range(3):
            acc = acc + _shift_w(sl, kx - 1) * wdw[ky * 3 + kx]
    y = jnp.dot(acc.reshape(HB3 * W, KK * C), wpw[...],
                preferred_element_type=jnp.float32) + bpw[0]
    out[...] = y


def _stage3(samp3, wdw, bdw, wpw, bpw):
    grid = H // HB3
    return pl.pallas_call(
        _stage3_body,
        grid=(grid,),
        in_specs=[
            pl.BlockSpec((HB3, W, KK * C), lambda r: (r, 0, 0)),
            pl.BlockSpec(memory_space=pl.ANY),
            pl.BlockSpec((KK, KK * C), lambda r: (0, 0)),
            pl.BlockSpec((1, KK * C), lambda r: (0, 0)),
            pl.BlockSpec((KK * C, F), lambda r: (0, 0)),
            pl.BlockSpec((1, F), lambda r: (0, 0)),
        ],
        out_specs=pl.BlockSpec((HB3 * W, F), lambda r: (r, 0)),
        out_shape=jax.ShapeDtypeStruct((H * W, F), jnp.float32),
        scratch_shapes=[
            pltpu.VMEM((2, W, KK * C), jnp.float32),
            pltpu.SemaphoreType.DMA,
        ],
    )(samp3, samp3, wdw, bdw, wpw, bpw)


def kernel(x, W_off, b_off, W_dw, b_dw, W_pw, b_pw):
    x2 = x.reshape(H, W, C)
    wf = W_off.reshape(KK, C, 2 * KK)
    wofft = jnp.concatenate([wf[:, :, 0::2], wf[:, :, 1::2]], axis=-1)
    boff2 = jnp.concatenate([b_off[0::2], b_off[1::2]]).reshape(1, 2 * KK)

    ia, ib, ic, id_, wa, wb, wc, wd, xpad = _stage1(x2, wofft, boff2)
    flat = lambda a: a.reshape(N)
    samp = _stage2(xpad, flat(ia), flat(ib), flat(ic), flat(id_),
                   flat(wa), flat(wb), flat(wc), flat(wd))

    samp3 = samp[:, :C].reshape(H, W, KK * C)
    wdw = W_dw.reshape(KK, KK * C)
    bdw = b_dw.reshape(1, KK * C)
    wpw = W_pw.reshape(KK * C, F)
    bpw = b_pw.reshape(1, F)
    y = _stage3(samp3, wdw, bdw, wpw, bpw)
    return y.reshape(1, H, W, F)


# speedup vs baseline: 4.9160x; 1.0511x over previous
import functools

import jax
import jax.numpy as jnp
from jax import lax
from jax.experimental import pallas as pl
from jax.experimental.pallas import tpu as pltpu
from jax.experimental.pallas import tpu_sc as plsc

H = 224
W = 224
C = 96
K = 3
KK = K * K
F = 96
N = H * W * KK

HB1 = 16
HB3 = 8
NW = 32
CP = 128
NR = N // 128
SB = HB1 * W * KK
SR = SB // 128
US = 64


def _copy_halo_rows(src_any, halo_ref, sem, r, hb, nrows):
    top = jnp.maximum(r * hb - 1, 0)
    bot = jnp.minimum(r * hb + hb, nrows - 1)
    cp0 = pltpu.make_async_copy(src_any.at[pl.ds(top, 1)], halo_ref.at[pl.ds(0, 1)], sem)
    cp0.start()
    cp0.wait()
    cp1 = pltpu.make_async_copy(src_any.at[pl.ds(bot, 1)], halo_ref.at[pl.ds(1, 1)], sem)
    cp1.start()
    cp1.wait()
    nblk = nrows // hb
    tmask = jnp.where(r == 0, 0.0, 1.0)
    bmask = jnp.where(r == nblk - 1, 0.0, 1.0)
    return tmask, bmask


def _shift_w(a, dx):
    hb, w, ch = a.shape
    z = jnp.zeros((hb, 1, ch), a.dtype)
    if dx == -1:
        return jnp.concatenate([z, a[:, : w - 1, :]], axis=1)
    if dx == 1:
        return jnp.concatenate([a[:, 1:, :], z], axis=1)
    return a


def _stage1_body(x_blk, x_any, wofft, boff2, ia, ib, ic, id_, wa, wb, wc, wd,
                 xpad, halo, sem):
    r = pl.program_id(0)
    tmask, bmask = _copy_halo_rows(x_any, halo, sem, r, HB1, H)
    xb = jnp.swapaxes(x_blk[...], 1, 2)
    top = (jnp.swapaxes(halo[0], 0, 1) * tmask)[None]
    bot = (jnp.swapaxes(halo[1], 0, 1) * bmask)[None]
    rows = jnp.concatenate([top, xb, bot], axis=0)

    off = jnp.zeros((HB1 * W, 2 * KK), jnp.float32)
    for ky in range(3):
        sl = rows[ky:ky + HB1]
        for kx in range(3):
            sh = _shift_w(sl, kx - 1).reshape(HB1 * W, C)
            off = off + jnp.dot(sh, wofft[ky * 3 + kx],
                                preferred_element_type=jnp.float32)
    off = off + boff2[0]
    off3 = off.reshape(HB1, W, 2 * KK)
    ox = off3[:, :, :KK]
    oy = off3[:, :, KK:]

    gx = lax.broadcasted_iota(jnp.int32, (HB1, W, KK), 1).astype(jnp.float32)
    gy = (lax.broadcasted_iota(jnp.int32, (HB1, W, KK), 0)
          + r * HB1).astype(jnp.float32)
    kkf = lax.broadcasted_iota(jnp.int32, (HB1, W, KK), 2).astype(jnp.float32)
    t = jnp.floor(kkf / 3.0)
    dyk = t - 1.0
    dxk = kkf - 3.0 * t - 1.0

    lx = jnp.clip(gx + dxk + ox, 0.0, W - 1.0)
    ly = jnp.clip(gy + dyk + oy, 0.0, H - 1.0)
    x0 = jnp.floor(lx)
    x1 = x0 + 1.0
    y0 = jnp.floor(ly)
    y1 = y0 + 1.0
    x0 = jnp.clip(x0, 0.0, W - 1.0)
    x1 = jnp.clip(x1, 0.0, W - 1.0)
    y0 = jnp.clip(y0, 0.0, H - 1.0)
    y1 = jnp.clip(y1, 0.0, H - 1.0)
    wa[...] = (x1 - lx) * (y1 - ly)
    wb[...] = (x1 - lx) * (ly - y0)
    wc[...] = (lx - x0) * (y1 - ly)
    wd[...] = (lx - x0) * (ly - y0)
    x0i = x0.astype(jnp.int32)
    x1i = x1.astype(jnp.int32)
    y0i = y0.astype(jnp.int32)
    y1i = y1.astype(jnp.int32)
    ia[...] = y0i * W + x0i
    ib[...] = y1i * W + x0i
    ic[...] = y0i * W + x1i
    id_[...] = y1i * W + x1i
    xpad[...] = jnp.concatenate(
        [xb.reshape(HB1 * W, C), jnp.zeros((HB1 * W, CP - C), jnp.float32)],
        axis=1)


def _stage1(x2, wofft, boff2):
    grid = H // HB1
    idx_like = jax.ShapeDtypeStruct((H, W, KK), jnp.int32)
    w_like = jax.ShapeDtypeStruct((H, W, KK), jnp.float32)
    out_spec = pl.BlockSpec((HB1, W, KK), lambda r: (r, 0, 0))
    return pl.pallas_call(
        _stage1_body,
        grid=(grid,),
        in_specs=[
            pl.BlockSpec((HB1, C, W), lambda r: (r, 0, 0)),
            pl.BlockSpec(memory_space=pl.ANY),
            pl.BlockSpec((KK, C, 2 * KK), lambda r: (0, 0, 0)),
            pl.BlockSpec((1, 2 * KK), lambda r: (0, 0)),
        ],
        out_specs=[out_spec] * 8 + [pl.BlockSpec((HB1 * W, CP), lambda r: (r, 0))],
        out_shape=[idx_like] * 4 + [w_like] * 4
        + [jax.ShapeDtypeStruct((H * W, CP), jnp.float32)],
        scratch_shapes=[
            pltpu.VMEM((2, C, W), jnp.float32),
            pltpu.SemaphoreType.DMA,
        ],
    )(x2, x2, wofft, boff2)


def _fire_gathers(xflat, pki, half, bufs, sem):
    for q, buf in enumerate(bufs):
        pltpu.make_async_copy(
            xflat.at[pki.at[q, pl.ds(half * US, US)]], buf, sem).start()


def _wait_gathers(xflat, pki, half, bufs, sem):
    for q, buf in enumerate(bufs):
        pltpu.make_async_copy(
            xflat.at[pki.at[q, pl.ds(half * US, US)]], buf, sem).wait()


def _row_copies(idx8, row, pki, pkw, sem):
    cps = []
    for q in range(4):
        cps.append(pltpu.make_async_copy(
            idx8[q].at[pl.ds(row * 128, 128)], pki.at[q], sem))
    for q in range(4):
        cps.append(pltpu.make_async_copy(
            idx8[4 + q].at[pl.ds(row * 128, 128)], pkw.at[q], sem))
    return cps


def _fetch_row(idx8, row, pki, pkw, sem):
    for cp in _row_copies(idx8, row, pki, pkw, sem):
        cp.start()


def _wait_row(idx8, row, pki, pkw, sem):
    for cp in _row_copies(idx8, row, pki, pkw, sem):
        cp.wait()


def _compute_unit(pkw, half, bufs, ob):
    a4, b4, c4, d4 = bufs

    def blk(tb, carry):
        b16 = tb * 16
        lane = pl.ds(half * US + tb * 16, 16)
        wav = pkw[0, lane]
        wbv = pkw[1, lane]
        wcv = pkw[2, lane]
        wdv = pkw[3, lane]
        for i in range(16):
            srow = b16 + i
            va = jnp.full((16,), wav[i], jnp.float32)
            vb = jnp.full((16,), wbv[i], jnp.float32)
            vc = jnp.full((16,), wcv[i], jnp.float32)
            vd = jnp.full((16,), wdv[i], jnp.float32)
            for cb in range(C // 16):
                slc = pl.ds(cb * 16, 16)
                ob[srow, slc] = (va * a4[srow, slc] + vb * b4[srow, slc]
                                 + vc * c4[srow, slc] + vd * d4[srow, slc])
        return carry

    lax.fori_loop(0, US // 16, blk, 0)


def _sc_body(row0, nrows, xflat, ia, ib, ic, id_, wa, wb, wc, wd, out,
             pi0, pi1, pw0, pw1, a0, b0, c0, d0, a1, b1, c1, d1, o0, o1,
             sgat0, sgat1, sout0, sout1, sidx0, sidx1):
    cid = lax.axis_index("c")
    sid = lax.axis_index("s")
    wid = sid * 2 + cid
    pt = nrows // 2
    base = pt // NW
    rem = pt % NW
    r0w = row0 + 2 * (base * wid + jnp.minimum(wid, rem))
    nr = 2 * (base + jnp.where(wid < rem, 1, 0))

    idx8 = (ia, ib, ic, id_, wa, wb, wc, wd)
    pki = (pi0, pi1)
    pkw = (pw0, pw1)
    gb = ((a0, b0, c0, d0), (a1, b1, c1, d1))
    ob = (o0, o1)
    sgat = (sgat0, sgat1)
    sout = (sout0, sout1)
    sidx = (sidx0, sidx1)

    def _zrow(i, carry):
        z = jnp.zeros((16,), jnp.float32)
        for buf in (o0, o1):
            buf[i, pl.ds(C, 16)] = z
            buf[i, pl.ds(C + 16, 16)] = z
        return carry
    lax.fori_loop(0, US, _zrow, 0)

    _fetch_row(idx8, r0w, pi0, pw0, sidx0)
    _wait_row(idx8, r0w, pi0, pw0, sidx0)
    _fire_gathers(xflat, pi0, 0, gb[0], sgat0)

    @pl.loop(0, nr, step=2)
    def _row_pair(rl):
        for rp in (0, 1):
            ar = r0w + rl + rp
            for b in (0, 1):
                _wait_gathers(xflat, pki[rp], b, gb[b], sgat[b])
                if b == 0:
                    @pl.when(rl + rp + 1 < nr)
                    def _():
                        _fetch_row(idx8, ar + 1, pki[rp ^ 1], pkw[rp ^ 1],
                                   sidx[rp ^ 1])
                    _fire_gathers(xflat, pki[rp], 1, gb[1], sgat[1])
                else:
                    @pl.when(rl + rp + 1 < nr)
                    def _():
                        _wait_row(idx8, ar + 1, pki[rp ^ 1], pkw[rp ^ 1],
                                  sidx[rp ^ 1])
                        _fire_gathers(xflat, pki[rp ^ 1], 0, gb[0], sgat[0])

                @pl.when(rl + rp >= 1)
                def _():
                    pltpu.make_async_copy(
                        ob[b], out.at[pl.ds(0, US)], sout[b]).wait()
                _compute_unit(pkw[rp], b, gb[b], ob[b])
                pltpu.make_async_copy(
                    ob[b],
                    out.at[pl.ds((ar - row0) * 128 + b * US, US)],
                    sout[b]).start()

    for b in (0, 1):
        pltpu.make_async_copy(ob[b], out.at[pl.ds(0, US)], sout[b]).wait()


def _stage2(xflat, ia, ib, ic, id_, wa, wb, wc, wd, row0, nrows):
    mesh = plsc.VectorSubcoreMesh(core_axis_name="c", subcore_axis_name="s")
    f = functools.partial(
        pl.kernel,
        out_type=jax.ShapeDtypeStruct((nrows * 128, CP), jnp.float32),
        mesh=mesh,
        scratch_types=[
            pltpu.VMEM((4, 128), jnp.int32),
            pltpu.VMEM((4, 128), jnp.int32),
            pltpu.VMEM((4, 128), jnp.float32),
            pltpu.VMEM((4, 128), jnp.float32),
            pltpu.VMEM((US, CP), jnp.float32),
            pltpu.VMEM((US, CP), jnp.float32),
            pltpu.VMEM((US, CP), jnp.float32),
            pltpu.VMEM((US, CP), jnp.float32),
            pltpu.VMEM((US, CP), jnp.float32),
            pltpu.VMEM((US, CP), jnp.float32),
            pltpu.VMEM((US, CP), jnp.float32),
            pltpu.VMEM((US, CP), jnp.float32),
            pltpu.VMEM((US, CP), jnp.float32),
            pltpu.VMEM((US, CP), jnp.float32),
            pltpu.SemaphoreType.DMA,
            pltpu.SemaphoreType.DMA,
            pltpu.SemaphoreType.DMA,
            pltpu.SemaphoreType.DMA,
            pltpu.SemaphoreType.DMA,
            pltpu.SemaphoreType.DMA,
        ],
    )(functools.partial(_sc_body, row0, nrows))
    return f(xflat, ia, ib, ic, id_, wa, wb, wc, wd)


def _stage3_body(blk0, top_edge, bot_edge, s_blk, s_any, wdw, bdw, wpw, bpw,
                  out, halo, sem):
    r = pl.program_id(0) + blk0
    nloc = s_any.shape[0] // HB3
    top = jnp.maximum(r * HB3 - 1, 0)
    bot = jnp.minimum(r * HB3 + HB3, nloc * HB3 - 1)
    cp0 = pltpu.make_async_copy(s_any.at[pl.ds(top, 1)], halo.at[pl.ds(0, 1)], sem)
    cp0.start()
    cp0.wait()
    cp1 = pltpu.make_async_copy(s_any.at[pl.ds(bot, 1)], halo.at[pl.ds(1, 1)], sem)
    cp1.start()
    cp1.wait()
    tmask = jnp.where(jnp.logical_and(r == 0, top_edge), 0.0, 1.0)
    bmask = jnp.where(jnp.logical_and(r == nloc - 1, bot_edge), 0.0, 1.0)
    sb = s_blk[...]
    top = (halo[0] * tmask)[None]
    bot = (halo[1] * bmask)[None]
    rows = jnp.concatenate([top, sb, bot], axis=0)

    acc = jnp.broadcast_to(bdw[0], (HB3, W, KK * C))
    for ky in range(3):
        sl = rows[ky:ky + HB3]
        for kx in range(3):
            acc = acc + _shift_w(sl, kx - 1) * wdw[ky * 3 + kx]
    y = jnp.dot(acc.reshape(HB3 * W, KK * C), wpw[...],
                preferred_element_type=jnp.float32) + bpw[0]
    out[...] = jnp.swapaxes(y.reshape(HB3, W, F), 1, 2)


def _stage3(samp3, wdw, bdw, wpw, bpw, blk0, nblk, top_edge, bot_edge):
    return pl.pallas_call(
        functools.partial(_stage3_body, blk0, top_edge, bot_edge),
        grid=(nblk,),
        in_specs=[
            pl.BlockSpec((HB3, W, KK * C), lambda r: (r + blk0, 0, 0)),
            pl.BlockSpec(memory_space=pl.ANY),
            pl.BlockSpec((KK, KK * C), lambda r: (0, 0)),
            pl.BlockSpec((1, KK * C), lambda r: (0, 0)),
            pl.BlockSpec((KK * C, F), lambda r: (0, 0)),
            pl.BlockSpec((1, F), lambda r: (0, 0)),
        ],
        out_specs=pl.BlockSpec((HB3, F, W), lambda r: (r, 0, 0)),
        out_shape=jax.ShapeDtypeStruct((nblk * HB3, F, W), jnp.float32),
        scratch_shapes=[
            pltpu.VMEM((2, W, KK * C), jnp.float32),
            pltpu.SemaphoreType.DMA,
        ],
    )(samp3, samp3, wdw, bdw, wpw, bpw)


def kernel(x, W_off, b_off, W_dw, b_dw, W_pw, b_pw):
    x2 = jnp.swapaxes(x.reshape(H, W, C), 1, 2)
    wf = W_off.reshape(KK, C, 2 * KK)
    wofft = jnp.concatenate([wf[:, :, 0::2], wf[:, :, 1::2]], axis=-1)
    boff2 = jnp.concatenate([b_off[0::2], b_off[1::2]]).reshape(1, 2 * KK)

    ia, ib, ic, id_, wa, wb, wc, wd, xpad = _stage1(x2, wofft, boff2)
    flat = lambda a: a.reshape(N)
    args8 = (flat(ia), flat(ib), flat(ic), flat(id_),
             flat(wa), flat(wb), flat(wc), flat(wd))
    RT = 1890
    s_top = _stage2(xpad, *args8, 0, RT)
    s_bot = _stage2(xpad, *args8, NR - RT, RT)

    wdw = W_dw.reshape(KK, KK * C)
    bdw = b_dw.reshape(1, KK * C)
    wpw = W_pw.reshape(KK * C, F)
    bpw = b_pw.reshape(1, F)
    v_top = s_top[:, :C].reshape(120, W, KK * C)
    v_bot = s_bot[:, :C].reshape(120, W, KK * C)
    y_top = _stage3(v_top, wdw, bdw, wpw, bpw, 0, 14, True, False)
    y_bot = _stage3(v_bot, wdw, bdw, wpw, bpw, 1, 14, False, True)
    y = jnp.concatenate([y_top, y_bot], axis=0)
    return jnp.swapaxes(y, 1, 2).reshape(1, H, W, F)


# speedup vs baseline: 5.0170x; 1.0205x over previous
import functools

import jax
import jax.numpy as jnp
from jax import lax
from jax.experimental import pallas as pl
from jax.experimental.pallas import tpu as pltpu
from jax.experimental.pallas import tpu_sc as plsc

H = 224
W = 224
C = 96
K = 3
KK = K * K
F = 96
N = H * W * KK

HB1 = 16
HB3 = 8
NW = 32
CP = 128
NR = N // 128
SB = HB1 * W * KK
SR = SB // 128
US = 64


def _copy_halo_rows(src_any, halo_ref, sem, r, hb, nrows):
    top = jnp.maximum(r * hb - 1, 0)
    bot = jnp.minimum(r * hb + hb, nrows - 1)
    cp0 = pltpu.make_async_copy(src_any.at[pl.ds(top, 1)], halo_ref.at[pl.ds(0, 1)], sem)
    cp0.start()
    cp0.wait()
    cp1 = pltpu.make_async_copy(src_any.at[pl.ds(bot, 1)], halo_ref.at[pl.ds(1, 1)], sem)
    cp1.start()
    cp1.wait()
    nblk = nrows // hb
    tmask = jnp.where(r == 0, 0.0, 1.0)
    bmask = jnp.where(r == nblk - 1, 0.0, 1.0)
    return tmask, bmask


def _shift_w(a, dx):
    hb, w, ch = a.shape
    z = jnp.zeros((hb, 1, ch), a.dtype)
    if dx == -1:
        return jnp.concatenate([z, a[:, : w - 1, :]], axis=1)
    if dx == 1:
        return jnp.concatenate([a[:, 1:, :], z], axis=1)
    return a


def _stage1_body(x_blk, x_any, wofft, boff2, ia, ib, ic, id_, wa, wb, wc, wd,
                 xpad, halo, sem):
    r = pl.program_id(0)
    tmask, bmask = _copy_halo_rows(x_any, halo, sem, r, HB1, H)
    xb = jnp.swapaxes(x_blk[...], 1, 2)
    top = (jnp.swapaxes(halo[0], 0, 1) * tmask)[None]
    bot = (jnp.swapaxes(halo[1], 0, 1) * bmask)[None]
    rows = jnp.concatenate([top, xb, bot], axis=0)

    off = jnp.zeros((HB1 * W, 2 * KK), jnp.float32)
    for ky in range(3):
        sl = rows[ky:ky + HB1]
        for kx in range(3):
            sh = _shift_w(sl, kx - 1).reshape(HB1 * W, C)
            off = off + jnp.dot(sh, wofft[ky * 3 + kx],
                                preferred_element_type=jnp.float32)
    off = off + boff2[0]
    off3 = off.reshape(HB1, W, 2 * KK)
    ox = off3[:, :, :KK]
    oy = off3[:, :, KK:]

    gx = lax.broadcasted_iota(jnp.int32, (HB1, W, KK), 1).astype(jnp.float32)
    gy = (lax.broadcasted_iota(jnp.int32, (HB1, W, KK), 0)
          + r * HB1).astype(jnp.float32)
    kkf = lax.broadcasted_iota(jnp.int32, (HB1, W, KK), 2).astype(jnp.float32)
    t = jnp.floor(kkf / 3.0)
    dyk = t - 1.0
    dxk = kkf - 3.0 * t - 1.0

    lx = jnp.clip(gx + dxk + ox, 0.0, W - 1.0)
    ly = jnp.clip(gy + dyk + oy, 0.0, H - 1.0)
    x0 = jnp.floor(lx)
    x1 = x0 + 1.0
    y0 = jnp.floor(ly)
    y1 = y0 + 1.0
    x0 = jnp.clip(x0, 0.0, W - 1.0)
    x1 = jnp.clip(x1, 0.0, W - 1.0)
    y0 = jnp.clip(y0, 0.0, H - 1.0)
    y1 = jnp.clip(y1, 0.0, H - 1.0)
    wa[...] = (x1 - lx) * (y1 - ly)
    wb[...] = (x1 - lx) * (ly - y0)
    wc[...] = (lx - x0) * (y1 - ly)
    wd[...] = (lx - x0) * (ly - y0)
    x0i = x0.astype(jnp.int32)
    x1i = x1.astype(jnp.int32)
    y0i = y0.astype(jnp.int32)
    y1i = y1.astype(jnp.int32)
    ia[...] = y0i * W + x0i
    ib[...] = y1i * W + x0i
    ic[...] = y0i * W + x1i
    id_[...] = y1i * W + x1i
    xpad[...] = jnp.concatenate(
        [xb.reshape(HB1 * W, C), jnp.zeros((HB1 * W, CP - C), jnp.float32)],
        axis=1)


def _stage1(x2, wofft, boff2):
    grid = H // HB1
    idx_like = jax.ShapeDtypeStruct((H, W, KK), jnp.int32)
    w_like = jax.ShapeDtypeStruct((H, W, KK), jnp.float32)
    out_spec = pl.BlockSpec((HB1, W, KK), lambda r: (r, 0, 0))
    return pl.pallas_call(
        _stage1_body,
        grid=(grid,),
        in_specs=[
            pl.BlockSpec((HB1, C, W), lambda r: (r, 0, 0)),
            pl.BlockSpec(memory_space=pl.ANY),
            pl.BlockSpec((KK, C, 2 * KK), lambda r: (0, 0, 0)),
            pl.BlockSpec((1, 2 * KK), lambda r: (0, 0)),
        ],
        out_specs=[out_spec] * 8 + [pl.BlockSpec((HB1 * W, CP), lambda r: (r, 0))],
        out_shape=[idx_like] * 4 + [w_like] * 4
        + [jax.ShapeDtypeStruct((H * W, CP), jnp.float32)],
        scratch_shapes=[
            pltpu.VMEM((2, C, W), jnp.float32),
            pltpu.SemaphoreType.DMA,
        ],
    )(x2, x2, wofft, boff2)


def _fire_gathers(xflat, pki, half, bufs, sem):
    for q, buf in enumerate(bufs):
        pltpu.make_async_copy(
            xflat.at[pki.at[q, pl.ds(half * US, US)]], buf, sem).start()


def _wait_gathers(xflat, pki, half, bufs, sem):
    for q, buf in enumerate(bufs):
        pltpu.make_async_copy(
            xflat.at[pki.at[q, pl.ds(half * US, US)]], buf, sem).wait()


def _row_copies(idx8, row, pki, pkw, sem):
    cps = []
    for q in range(4):
        cps.append(pltpu.make_async_copy(
            idx8[q].at[pl.ds(row * 128, 128)], pki.at[q], sem))
    for q in range(4):
        cps.append(pltpu.make_async_copy(
            idx8[4 + q].at[pl.ds(row * 128, 128)], pkw.at[q], sem))
    return cps


def _fetch_row(idx8, row, pki, pkw, sem):
    for cp in _row_copies(idx8, row, pki, pkw, sem):
        cp.start()


def _wait_row(idx8, row, pki, pkw, sem):
    for cp in _row_copies(idx8, row, pki, pkw, sem):
        cp.wait()


def _compute_unit(pkw, half, bufs, ob):
    a4, b4, c4, d4 = bufs

    def blk(tb, carry):
        b16 = tb * 16
        lane = pl.ds(half * US + tb * 16, 16)
        wav = pkw[0, lane]
        wbv = pkw[1, lane]
        wcv = pkw[2, lane]
        wdv = pkw[3, lane]
        for i in range(16):
            srow = b16 + i
            va = jnp.full((16,), wav[i], jnp.float32)
            vb = jnp.full((16,), wbv[i], jnp.float32)
            vc = jnp.full((16,), wcv[i], jnp.float32)
            vd = jnp.full((16,), wdv[i], jnp.float32)
            for cb in range(C // 16):
                slc = pl.ds(cb * 16, 16)
                ob[srow, slc] = (va * a4[srow, slc] + vb * b4[srow, slc]
                                 + vc * c4[srow, slc] + vd * d4[srow, slc])
        return carry

    lax.fori_loop(0, US // 16, blk, 0)


def _sc_body(xflat, ia, ib, ic, id_, wa, wb, wc, wd, out,
             pi0, pi1, pw0, pw1, a0, b0, c0, d0, a1, b1, c1, d1, o0, o1,
             sgat0, sgat1, sout0, sout1, sidx0, sidx1):
    cid = lax.axis_index("c")
    sid = lax.axis_index("s")
    wid = sid * 2 + cid
    r0w = wid * 110 + 2 * jnp.minimum(wid, 4)
    nr = jnp.where(wid < 4, 112, 110)

    idx8 = (ia, ib, ic, id_, wa, wb, wc, wd)
    pki = (pi0, pi1)
    pkw = (pw0, pw1)
    gb = ((a0, b0, c0, d0), (a1, b1, c1, d1))
    ob = (o0, o1)
    sgat = (sgat0, sgat1)
    sout = (sout0, sout1)
    sidx = (sidx0, sidx1)

    def _zrow(i, carry):
        z = jnp.zeros((16,), jnp.float32)
        for buf in (o0, o1):
            buf[i, pl.ds(C, 16)] = z
            buf[i, pl.ds(C + 16, 16)] = z
        return carry
    lax.fori_loop(0, US, _zrow, 0)

    _fetch_row(idx8, r0w, pi0, pw0, sidx0)
    _wait_row(idx8, r0w, pi0, pw0, sidx0)
    _fire_gathers(xflat, pi0, 0, gb[0], sgat0)

    @pl.loop(0, nr, step=2)
    def _row_pair(rl):
        for rp in (0, 1):
            ar = r0w + rl + rp
            for b in (0, 1):
                _wait_gathers(xflat, pki[rp], b, gb[b], sgat[b])
                if b == 0:
                    @pl.when(rl + rp + 1 < nr)
                    def _():
                        _fetch_row(idx8, ar + 1, pki[rp ^ 1], pkw[rp ^ 1],
                                   sidx[rp ^ 1])
                    _fire_gathers(xflat, pki[rp], 1, gb[1], sgat[1])
                else:
                    @pl.when(rl + rp + 1 < nr)
                    def _():
                        _wait_row(idx8, ar + 1, pki[rp ^ 1], pkw[rp ^ 1],
                                  sidx[rp ^ 1])
                        _fire_gathers(xflat, pki[rp ^ 1], 0, gb[0], sgat[0])

                @pl.when(rl + rp >= 1)
                def _():
                    pltpu.make_async_copy(
                        ob[b], out.at[pl.ds(0, US)], sout[b]).wait()
                _compute_unit(pkw[rp], b, gb[b], ob[b])
                pltpu.make_async_copy(
                    ob[b], out.at[pl.ds(ar * 128 + b * US, US)], sout[b]).start()

    for b in (0, 1):
        pltpu.make_async_copy(ob[b], out.at[pl.ds(0, US)], sout[b]).wait()


def _stage2(xflat, ia, ib, ic, id_, wa, wb, wc, wd):
    mesh = plsc.VectorSubcoreMesh(core_axis_name="c", subcore_axis_name="s")
    f = functools.partial(
        pl.kernel,
        out_type=jax.ShapeDtypeStruct((N, CP), jnp.float32),
        mesh=mesh,
        scratch_types=[
            pltpu.VMEM((4, 128), jnp.int32),
            pltpu.VMEM((4, 128), jnp.int32),
            pltpu.VMEM((4, 128), jnp.float32),
            pltpu.VMEM((4, 128), jnp.float32),
            pltpu.VMEM((US, CP), jnp.float32),
            pltpu.VMEM((US, CP), jnp.float32),
            pltpu.VMEM((US, CP), jnp.float32),
            pltpu.VMEM((US, CP), jnp.float32),
            pltpu.VMEM((US, CP), jnp.float32),
            pltpu.VMEM((US, CP), jnp.float32),
            pltpu.VMEM((US, CP), jnp.float32),
            pltpu.VMEM((US, CP), jnp.float32),
            pltpu.VMEM((US, CP), jnp.float32),
            pltpu.VMEM((US, CP), jnp.float32),
            pltpu.SemaphoreType.DMA,
            pltpu.SemaphoreType.DMA,
            pltpu.SemaphoreType.DMA,
            pltpu.SemaphoreType.DMA,
            pltpu.SemaphoreType.DMA,
            pltpu.SemaphoreType.DMA,
        ],
    )(_sc_body)
    return f(xflat, ia, ib, ic, id_, wa, wb, wc, wd)


def _stage3_body(s_blk, s_any, wdw, bdw, wpw, bpw, out, halo, sem):
    r = pl.program_id(0)
    tmask, bmask = _copy_halo_rows(s_any, halo, sem, r, HB3, H)
    sb = s_blk[...]
    top = (halo[0] * tmask)[None]
    bot = (halo[1] * bmask)[None]
    rows = jnp.concatenate([top, sb, bot], axis=0)

    acc = jnp.broadcast_to(bdw[0], (HB3, W, KK * C))
    for ky in range(3):
        sl = rows[ky:ky + HB3]
        for kx in range(3):
            acc = acc + _shift_w(sl, kx - 1) * wdw[ky * 3 + kx]
    y = jnp.dot(acc.reshape(HB3 * W, KK * C), wpw[...],
                preferred_element_type=jnp.float32) + bpw[0]
    out[...] = jnp.swapaxes(y.reshape(HB3, W, F), 1, 2)


def _stage3(samp3, wdw, bdw, wpw, bpw):
    grid = H // HB3
    return pl.pallas_call(
        _stage3_body,
        grid=(grid,),
        in_specs=[
            pl.BlockSpec((HB3, W, KK * C), lambda r: (r, 0, 0)),
            pl.BlockSpec(memory_space=pl.ANY),
            pl.BlockSpec((KK, KK * C), lambda r: (0, 0)),
            pl.BlockSpec((1, KK * C), lambda r: (0, 0)),
            pl.BlockSpec((KK * C, F), lambda r: (0, 0)),
            pl.BlockSpec((1, F), lambda r: (0, 0)),
        ],
        out_specs=pl.BlockSpec((HB3, F, W), lambda r: (r, 0, 0)),
        out_shape=jax.ShapeDtypeStruct((H, F, W), jnp.float32),
        scratch_shapes=[
            pltpu.VMEM((2, W, KK * C), jnp.float32),
            pltpu.SemaphoreType.DMA,
        ],
    )(samp3, samp3, wdw, bdw, wpw, bpw)


def kernel(x, W_off, b_off, W_dw, b_dw, W_pw, b_pw):
    x2 = jnp.swapaxes(x.reshape(H, W, C), 1, 2)
    wf = W_off.reshape(KK, C, 2 * KK)
    wofft = jnp.concatenate([wf[:, :, 0::2], wf[:, :, 1::2]], axis=-1)
    boff2 = jnp.concatenate([b_off[0::2], b_off[1::2]]).reshape(1, 2 * KK)

    ia, ib, ic, id_, wa, wb, wc, wd, xpad = _stage1(x2, wofft, boff2)
    flat = lambda a: a.reshape(N)
    samp = _stage2(xpad, flat(ia), flat(ib), flat(ic), flat(id_),
                   flat(wa), flat(wb), flat(wc), flat(wd))

    samp3 = samp[:, :C].reshape(H, W, KK * C)
    wdw = W_dw.reshape(KK, KK * C)
    bdw = b_dw.reshape(1, KK * C)
    wpw = W_pw.reshape(KK * C, F)
    bpw = b_pw.reshape(1, F)
    y = _stage3(samp3, wdw, bdw, wpw, bpw)
    return jnp.swapaxes(y, 1, 2).reshape(1, H, W, F)


# speedup vs baseline: 5.1526x; 1.0270x over previous
import functools

import jax
import jax.numpy as jnp
from jax import lax
from jax.experimental import pallas as pl
from jax.experimental.pallas import tpu as pltpu
from jax.experimental.pallas import tpu_sc as plsc

H = 224
W = 224
C = 96
K = 3
KK = K * K
F = 96
N = H * W * KK

HB1 = 16
HB3 = 16
NW = 32
CP = 128
NR = N // 128
SB = HB1 * W * KK
SR = SB // 128
US = 64


def _copy_halo_rows(src_any, halo_ref, sem, r, hb, nrows):
    top = jnp.maximum(r * hb - 1, 0)
    bot = jnp.minimum(r * hb + hb, nrows - 1)
    cp0 = pltpu.make_async_copy(src_any.at[pl.ds(top, 1)], halo_ref.at[pl.ds(0, 1)], sem)
    cp0.start()
    cp0.wait()
    cp1 = pltpu.make_async_copy(src_any.at[pl.ds(bot, 1)], halo_ref.at[pl.ds(1, 1)], sem)
    cp1.start()
    cp1.wait()
    nblk = nrows // hb
    tmask = jnp.where(r == 0, 0.0, 1.0)
    bmask = jnp.where(r == nblk - 1, 0.0, 1.0)
    return tmask, bmask


def _shift_w(a, dx):
    hb, w, ch = a.shape
    z = jnp.zeros((hb, 1, ch), a.dtype)
    if dx == -1:
        return jnp.concatenate([z, a[:, : w - 1, :]], axis=1)
    if dx == 1:
        return jnp.concatenate([a[:, 1:, :], z], axis=1)
    return a


def _stage1_body(x_blk, x_any, wofft, boff2, ia, ib, ic, id_, wa, wb, wc, wd,
                 xpad, halo, sem):
    r = pl.program_id(0)
    tmask, bmask = _copy_halo_rows(x_any, halo, sem, r, HB1, H)
    xb = jnp.swapaxes(x_blk[...], 1, 2)
    top = (jnp.swapaxes(halo[0], 0, 1) * tmask)[None]
    bot = (jnp.swapaxes(halo[1], 0, 1) * bmask)[None]
    rows = jnp.concatenate([top, xb, bot], axis=0)

    off = jnp.zeros((HB1 * W, 2 * KK), jnp.float32)
    for ky in range(3):
        sl = rows[ky:ky + HB1]
        for kx in range(3):
            sh = _shift_w(sl, kx - 1).reshape(HB1 * W, C)
            off = off + jnp.dot(sh, wofft[ky * 3 + kx],
                                preferred_element_type=jnp.float32)
    off = off + boff2[0]
    off3 = off.reshape(HB1, W, 2 * KK)
    ox = off3[:, :, :KK]
    oy = off3[:, :, KK:]

    gx = lax.broadcasted_iota(jnp.int32, (HB1, W, KK), 1).astype(jnp.float32)
    gy = (lax.broadcasted_iota(jnp.int32, (HB1, W, KK), 0)
          + r * HB1).astype(jnp.float32)
    kkf = lax.broadcasted_iota(jnp.int32, (HB1, W, KK), 2).astype(jnp.float32)
    t = jnp.floor(kkf / 3.0)
    dyk = t - 1.0
    dxk = kkf - 3.0 * t - 1.0

    lx = jnp.clip(gx + dxk + ox, 0.0, W - 1.0)
    ly = jnp.clip(gy + dyk + oy, 0.0, H - 1.0)
    x0 = jnp.floor(lx)
    x1 = x0 + 1.0
    y0 = jnp.floor(ly)
    y1 = y0 + 1.0
    x0 = jnp.clip(x0, 0.0, W - 1.0)
    x1 = jnp.clip(x1, 0.0, W - 1.0)
    y0 = jnp.clip(y0, 0.0, H - 1.0)
    y1 = jnp.clip(y1, 0.0, H - 1.0)
    wa[...] = (x1 - lx) * (y1 - ly)
    wb[...] = (x1 - lx) * (ly - y0)
    wc[...] = (lx - x0) * (y1 - ly)
    wd[...] = (lx - x0) * (ly - y0)
    x0i = x0.astype(jnp.int32)
    x1i = x1.astype(jnp.int32)
    y0i = y0.astype(jnp.int32)
    y1i = y1.astype(jnp.int32)
    ia[...] = y0i * W + x0i
    ib[...] = y1i * W + x0i
    ic[...] = y0i * W + x1i
    id_[...] = y1i * W + x1i
    xpad[...] = jnp.concatenate(
        [xb.reshape(HB1 * W, C), jnp.zeros((HB1 * W, CP - C), jnp.float32)],
        axis=1)


def _stage1(x2, wofft, boff2):
    grid = H // HB1
    idx_like = jax.ShapeDtypeStruct((H, W, KK), jnp.int32)
    w_like = jax.ShapeDtypeStruct((H, W, KK), jnp.float32)
    out_spec = pl.BlockSpec((HB1, W, KK), lambda r: (r, 0, 0))
    return pl.pallas_call(
        _stage1_body,
        grid=(grid,),
        in_specs=[
            pl.BlockSpec((HB1, C, W), lambda r: (r, 0, 0)),
            pl.BlockSpec(memory_space=pl.ANY),
            pl.BlockSpec((KK, C, 2 * KK), lambda r: (0, 0, 0)),
            pl.BlockSpec((1, 2 * KK), lambda r: (0, 0)),
        ],
        out_specs=[out_spec] * 8 + [pl.BlockSpec((HB1 * W, CP), lambda r: (r, 0))],
        out_shape=[idx_like] * 4 + [w_like] * 4
        + [jax.ShapeDtypeStruct((H * W, CP), jnp.float32)],
        scratch_shapes=[
            pltpu.VMEM((2, C, W), jnp.float32),
            pltpu.SemaphoreType.DMA,
        ],
    )(x2, x2, wofft, boff2)


def _fire_gathers(xflat, pki, half, bufs, sem):
    for q, buf in enumerate(bufs):
        pltpu.make_async_copy(
            xflat.at[pki.at[q, pl.ds(half * US, US)]], buf, sem).start()


def _wait_gathers(xflat, pki, half, bufs, sem):
    for q, buf in enumerate(bufs):
        pltpu.make_async_copy(
            xflat.at[pki.at[q, pl.ds(half * US, US)]], buf, sem).wait()


def _row_copies(idx8, row, pki, pkw, sem):
    cps = []
    for q in range(4):
        cps.append(pltpu.make_async_copy(
            idx8[q].at[pl.ds(row * 128, 128)], pki.at[q], sem))
    for q in range(4):
        cps.append(pltpu.make_async_copy(
            idx8[4 + q].at[pl.ds(row * 128, 128)], pkw.at[q], sem))
    return cps


def _fetch_row(idx8, row, pki, pkw, sem):
    for cp in _row_copies(idx8, row, pki, pkw, sem):
        cp.start()


def _wait_row(idx8, row, pki, pkw, sem):
    for cp in _row_copies(idx8, row, pki, pkw, sem):
        cp.wait()


def _compute_unit(pkw, half, bufs, ob):
    a4, b4, c4, d4 = bufs

    def blk(tb, carry):
        b16 = tb * 16
        lane = pl.ds(half * US + tb * 16, 16)
        wav = pkw[0, lane]
        wbv = pkw[1, lane]
        wcv = pkw[2, lane]
        wdv = pkw[3, lane]
        for i in range(16):
            srow = b16 + i
            va = jnp.full((16,), wav[i], jnp.float32)
            vb = jnp.full((16,), wbv[i], jnp.float32)
            vc = jnp.full((16,), wcv[i], jnp.float32)
            vd = jnp.full((16,), wdv[i], jnp.float32)
            for cb in range(C // 16):
                slc = pl.ds(cb * 16, 16)
                ob[srow, slc] = (va * a4[srow, slc] + vb * b4[srow, slc]
                                 + vc * c4[srow, slc] + vd * d4[srow, slc])
        return carry

    lax.fori_loop(0, US // 16, blk, 0)


def _sc_body(xflat, ia, ib, ic, id_, wa, wb, wc, wd, out,
             pi0, pi1, pw0, pw1, a0, b0, c0, d0, a1, b1, c1, d1, o0, o1,
             sgat0, sgat1, sout0, sout1, sidx0, sidx1):
    cid = lax.axis_index("c")
    sid = lax.axis_index("s")
    wid = sid * 2 + cid
    r0w = wid * 110 + 2 * jnp.minimum(wid, 4)
    nr = jnp.where(wid < 4, 112, 110)

    idx8 = (ia, ib, ic, id_, wa, wb, wc, wd)
    pki = (pi0, pi1)
    pkw = (pw0, pw1)
    gb = ((a0, b0, c0, d0), (a1, b1, c1, d1))
    ob = (o0, o1)
    sgat = (sgat0, sgat1)
    sout = (sout0, sout1)
    sidx = (sidx0, sidx1)

    def _zrow(i, carry):
        z = jnp.zeros((16,), jnp.float32)
        for buf in (o0, o1):
            buf[i, pl.ds(C, 16)] = z
            buf[i, pl.ds(C + 16, 16)] = z
        return carry
    lax.fori_loop(0, US, _zrow, 0)

    _fetch_row(idx8, r0w, pi0, pw0, sidx0)
    _wait_row(idx8, r0w, pi0, pw0, sidx0)
    _fire_gathers(xflat, pi0, 0, gb[0], sgat0)

    @pl.loop(0, nr, step=2)
    def _row_pair(rl):
        for rp in (0, 1):
            ar = r0w + rl + rp
            for b in (0, 1):
                _wait_gathers(xflat, pki[rp], b, gb[b], sgat[b])
                if b == 0:
                    @pl.when(rl + rp + 1 < nr)
                    def _():
                        _fetch_row(idx8, ar + 1, pki[rp ^ 1], pkw[rp ^ 1],
                                   sidx[rp ^ 1])
                    _fire_gathers(xflat, pki[rp], 1, gb[1], sgat[1])
                else:
                    @pl.when(rl + rp + 1 < nr)
                    def _():
                        _wait_row(idx8, ar + 1, pki[rp ^ 1], pkw[rp ^ 1],
                                  sidx[rp ^ 1])
                        _fire_gathers(xflat, pki[rp ^ 1], 0, gb[0], sgat[0])

                @pl.when(rl + rp >= 1)
                def _():
                    pltpu.make_async_copy(
                        ob[b], out.at[pl.ds(0, US)], sout[b]).wait()
                _compute_unit(pkw[rp], b, gb[b], ob[b])
                pltpu.make_async_copy(
                    ob[b], out.at[pl.ds(ar * 128 + b * US, US)], sout[b]).start()

    for b in (0, 1):
        pltpu.make_async_copy(ob[b], out.at[pl.ds(0, US)], sout[b]).wait()


def _stage2(xflat, ia, ib, ic, id_, wa, wb, wc, wd):
    mesh = plsc.VectorSubcoreMesh(core_axis_name="c", subcore_axis_name="s")
    f = functools.partial(
        pl.kernel,
        out_type=jax.ShapeDtypeStruct((N, CP), jnp.float32),
        mesh=mesh,
        scratch_types=[
            pltpu.VMEM((4, 128), jnp.int32),
            pltpu.VMEM((4, 128), jnp.int32),
            pltpu.VMEM((4, 128), jnp.float32),
            pltpu.VMEM((4, 128), jnp.float32),
            pltpu.VMEM((US, CP), jnp.float32),
            pltpu.VMEM((US, CP), jnp.float32),
            pltpu.VMEM((US, CP), jnp.float32),
            pltpu.VMEM((US, CP), jnp.float32),
            pltpu.VMEM((US, CP), jnp.float32),
            pltpu.VMEM((US, CP), jnp.float32),
            pltpu.VMEM((US, CP), jnp.float32),
            pltpu.VMEM((US, CP), jnp.float32),
            pltpu.VMEM((US, CP), jnp.float32),
            pltpu.VMEM((US, CP), jnp.float32),
            pltpu.SemaphoreType.DMA,
            pltpu.SemaphoreType.DMA,
            pltpu.SemaphoreType.DMA,
            pltpu.SemaphoreType.DMA,
            pltpu.SemaphoreType.DMA,
            pltpu.SemaphoreType.DMA,
        ],
    )(_sc_body)
    return f(xflat, ia, ib, ic, id_, wa, wb, wc, wd)


def _stage3_body(s_blk, s_any, wdw, bdw, wpw, bpw, out, halo, sem):
    r = pl.program_id(0)
    tmask, bmask = _copy_halo_rows(s_any, halo, sem, r, HB3, H)
    sb = s_blk[...]
    top = (halo[0] * tmask)[None]
    bot = (halo[1] * bmask)[None]
    rows = jnp.concatenate([top, sb, bot], axis=0)

    acc = jnp.broadcast_to(bdw[0], (HB3, W, KK * C))
    for ky in range(3):
        sl = rows[ky:ky + HB3]
        for kx in range(3):
            acc = acc + _shift_w(sl, kx - 1) * wdw[ky * 3 + kx]
    y = jnp.dot(acc.reshape(HB3 * W, KK * C), wpw[...],
                preferred_element_type=jnp.float32) + bpw[0]
    out[...] = jnp.swapaxes(y.reshape(HB3, W, F), 1, 2)


def _stage3(samp3, wdw, bdw, wpw, bpw):
    grid = H // HB3
    return pl.pallas_call(
        _stage3_body,
        grid=(grid,),
        in_specs=[
            pl.BlockSpec((HB3, W, KK * C), lambda r: (r, 0, 0)),
            pl.BlockSpec(memory_space=pl.ANY),
            pl.BlockSpec((KK, KK * C), lambda r: (0, 0)),
            pl.BlockSpec((1, KK * C), lambda r: (0, 0)),
            pl.BlockSpec((KK * C, F), lambda r: (0, 0)),
            pl.BlockSpec((1, F), lambda r: (0, 0)),
        ],
        out_specs=pl.BlockSpec((HB3, F, W), lambda r: (r, 0, 0)),
        out_shape=jax.ShapeDtypeStruct((H, F, W), jnp.float32),
        scratch_shapes=[
            pltpu.VMEM((2, W, KK * C), jnp.float32),
            pltpu.SemaphoreType.DMA,
        ],
    )(samp3, samp3, wdw, bdw, wpw, bpw)


def kernel(x, W_off, b_off, W_dw, b_dw, W_pw, b_pw):
    x2 = jnp.swapaxes(x.reshape(H, W, C), 1, 2)
    wf = W_off.reshape(KK, C, 2 * KK)
    wofft = jnp.concatenate([wf[:, :, 0::2], wf[:, :, 1::2]], axis=-1)
    boff2 = jnp.concatenate([b_off[0::2], b_off[1::2]]).reshape(1, 2 * KK)

    ia, ib, ic, id_, wa, wb, wc, wd, xpad = _stage1(x2, wofft, boff2)
    flat = lambda a: a.reshape(N)
    samp = _stage2(xpad, flat(ia), flat(ib), flat(ic), flat(id_),
                   flat(wa), flat(wb), flat(wc), flat(wd))

    samp3 = samp[:, :C].reshape(H, W, KK * C)
    wdw = W_dw.reshape(KK, KK * C)
    bdw = b_dw.reshape(1, KK * C)
    wpw = W_pw.reshape(KK * C, F)
    bpw = b_pw.reshape(1, F)
    y = _stage3(samp3, wdw, bdw, wpw, bpw)
    return jnp.swapaxes(y, 1, 2).reshape(1, H, W, F)
